# Initial kernel scaffold; baseline (speedup 1.0000x reference)
import jax
import jax.numpy as jnp
from jax.experimental import pallas as pl


def kernel(x, w1, b1, bn_scale, bn_shift, w2, b2):
    raise NotImplementedError("write your pallas kernel here")


# speedup vs baseline: 1.0336x; 1.0336x over previous
import functools

import jax
import jax.numpy as jnp
from jax.experimental import pallas as pl
from jax.experimental.pallas import tpu as pltpu


def _pick_c_tile(C):
    if C % 128 != 0:
        return C
    best = 128
    tc = 128
    while tc <= min(C, 512):
        if C % tc == 0:
            best = tc
        tc += 128
    return best


def _pick_b_tile(B):
    if B % 16 == 0:
        return B // 2
    return B


def _fused_kernel(x_ref, w1_ref, b1_ref, s_ref, t_ref, w2_ref, b2_ref,
                  out_ref, acc_ref, *, inv_hw, n_c):
    j = pl.program_id(1)
    feat = jnp.sum(x_ref[...], axis=-1, dtype=jnp.float32) * inv_hw
    part = jnp.dot(feat, w1_ref[...], preferred_element_type=jnp.float32)

    @pl.when(j == 0)
    def _():
        acc_ref[...] = part

    @pl.when(j > 0)
    def _():
        acc_ref[...] += part

    @pl.when(j == n_c - 1)
    def _():
        h = acc_ref[...] + b1_ref[...]
        h = jnp.maximum(h * s_ref[...] + t_ref[...], 0.0)
        scores = jnp.dot(h, w2_ref[...], preferred_element_type=jnp.float32)
        out_ref[...] = scores + b2_ref[...]


@jax.jit
def _forward(x, w1, b1, bn_scale, bn_shift, w2, b2):
    B, C, H, W = x.shape
    HW = H * W
    D1 = w1.shape[1]
    NC = w2.shape[1]
    xv = x.reshape(B, C, HW)
    tile_c = _pick_c_tile(C)
    tile_b = _pick_b_tile(B)
    n_c = C // tile_c
    grid = (B // tile_b, n_c)
    body = functools.partial(_fused_kernel, inv_hw=1.0 / float(HW), n_c=n_c)
    return pl.pallas_call(
        body,
        grid=grid,
        in_specs=[
            pl.BlockSpec((tile_b, tile_c, HW), lambda i, j: (i, j, 0)),
            pl.BlockSpec((tile_c, D1), lambda i, j: (j, 0)),
            pl.BlockSpec((1, D1), lambda i, j: (0, 0)),
            pl.BlockSpec((1, D1), lambda i, j: (0, 0)),
            pl.BlockSpec((1, D1), lambda i, j: (0, 0)),
            pl.BlockSpec((D1, NC), lambda i, j: (0, 0)),
            pl.BlockSpec((1, NC), lambda i, j: (0, 0)),
        ],
        out_specs=pl.BlockSpec((tile_b, NC), lambda i, j: (i, 0)),
        out_shape=jax.ShapeDtypeStruct((B, NC), jnp.float32),
        scratch_shapes=[pltpu.VMEM((tile_b, D1), jnp.float32)],
        compiler_params=pltpu.CompilerParams(
            dimension_semantics=("parallel", "arbitrary")),
        cost_estimate=pl.CostEstimate(
            flops=B * C * HW + 2 * B * C * D1 + 2 * B * D1 * NC,
            transcendentals=0,
            bytes_accessed=(B * C * HW * 4 + C * D1 * 4 + 3 * D1 * 4
                            + D1 * NC * 4 + NC * 4 + B * NC * 4)),
    )(xv, w1, b1, bn_scale, bn_shift, w2, b2)


def kernel(x, w1, b1, bn_scale, bn_shift, w2, b2):
    return _forward(x, w1, b1, bn_scale, bn_shift, w2, b2)
